# baseline (speedup 1.0000x reference)
import functools

import jax
import jax.numpy as jnp
from jax import lax
from jax.experimental import pallas as pl
from jax.experimental.pallas import tpu as pltpu
from jax.experimental.pallas import tpu_sc as plsc

_T = 1000
_N = 32
_NSTATE = 64
_NTRANS = 5
_C = _NSTATE * _NTRANS
_CH = 125
_NCHUNK = _T // _CH
_NORM_EVERY = 5
_LN2 = 0.6931471805599453


def _pgather(v, idxv):
    return v.at[idxv].get(mode="promise_in_bounds")


def _lane_bcast_max(v, iota):
    for sh in (8, 4, 2, 1):
        rot = (iota + sh) & 15
        v = jnp.maximum(v, _pgather(v, rot))
    return v


def _lane_bcast_sum(v, iota):
    for sh in (8, 4, 2, 1):
        rot = (iota + sh) & 15
        v = v + _pgather(v, rot)
    return v


def _sc_scan(scores_hbm, out_hbm, buf, pbuf, out_v, sem0, sem1):
    wid = lax.axis_index("c") * 16 + lax.axis_index("s")
    iota = lax.iota(jnp.int32, 16)
    gidx = [4 * iota + b for b in range(4)]
    colv = [
        [5 * (16 * (iota & 3) + 4 * (iota >> 2) + g) + j for j in range(_NTRANS)]
        for g in range(4)
    ]

    sems = (sem0, sem1)

    def chunk_src(c):
        return scores_hbm.at[pl.ds(c * _CH, _CH), wid]

    def step(t, p, mb, pbuf):
        tv = jnp.full((16,), t, jnp.int32)
        gath = [plsc.load_gather(pbuf, [gidx[b]]) for b in range(4)]
        newp = []
        for g in range(4):
            acc = None
            for j in range(_NTRANS):
                m = plsc.load_gather(mb, [tv, colv[g][j]])
                pv = p[g] if j == 0 else gath[j - 1]
                term = jnp.exp(m) * pv
                acc = term if acc is None else acc + term
            newp.append(acc)
        for g in range(4):
            pbuf[pl.ds(16 * g, 16)] = newp[g]
        return newp

    def renorm(p, cvec, pbuf):
        m = jnp.maximum(jnp.maximum(p[0], p[1]), jnp.maximum(p[2], p[3]))
        m = _lane_bcast_max(m, iota)
        k = (plsc.bitcast(m, jnp.int32) >> 23) - 127
        scale = plsc.bitcast((127 - k) << 23, jnp.float32)
        p = [pg * scale for pg in p]
        for g in range(4):
            pbuf[pl.ds(16 * g, 16)] = p[g]
        return p, cvec + k

    def body(i, carry, mb, pbuf):
        p = list(carry[:4])
        cvec = carry[4]
        for u in range(_NORM_EVERY):
            p = step(i * _NORM_EVERY + u, p, mb, pbuf)
        p, cvec = renorm(p, cvec, pbuf)
        return (*p, cvec)

    ones = jnp.ones((16,), jnp.float32)
    for g in range(4):
        pbuf[pl.ds(16 * g, 16)] = ones
    carry = (ones, ones, ones, ones, jnp.zeros((16,), jnp.int32))

    cp = pltpu.async_copy(chunk_src(0), buf.at[0], sems[0])
    for c in range(_NCHUNK):
        nxt = None
        if c + 1 < _NCHUNK:
            nxt = pltpu.async_copy(
                chunk_src(c + 1), buf.at[(c + 1) % 2], sems[(c + 1) % 2]
            )
        cp.wait()
        mb = buf.at[c % 2]
        carry = lax.fori_loop(
            0, _CH // _NORM_EVERY,
            functools.partial(body, mb=mb, pbuf=pbuf), carry, unroll=False,
        )
        cp = nxt

    p = carry[:4]
    cvec = carry[4]
    s = (p[0] + p[1]) + (p[2] + p[3])
    s = _lane_bcast_sum(s, iota)
    k2 = (plsc.bitcast(s, jnp.int32) >> 23) - 127
    mant = s * plsc.bitcast((127 - k2) << 23, jnp.float32)
    u = mant - 1.0
    z = u / (u + 2.0)
    z2 = z * z
    y = 2.0 * z * (1.0 + z2 * (1.0 / 3.0 + z2 * (0.2 + z2 * (1.0 / 7.0))))
    y = y + mant * jnp.exp(-y) - 1.0
    y = y + mant * jnp.exp(-y) - 1.0
    logz = (cvec + k2).astype(jnp.float32) * _LN2 + y
    out_v[...] = logz
    pltpu.sync_copy(out_v, out_hbm.at[pl.ds(wid * 16, 16)])


_sc_kernel = pl.kernel(
    _sc_scan,
    out_type=jax.ShapeDtypeStruct((_N * 16,), jnp.float32),
    mesh=plsc.VectorSubcoreMesh(core_axis_name="c", subcore_axis_name="s"),
    compiler_params=pltpu.CompilerParams(
        use_tc_tiling_on_sc=True, needs_layout_passes=False
    ),
    scratch_types=[
        pltpu.VMEM((2, _CH, _C), jnp.float32),
        pltpu.VMEM((_NSTATE,), jnp.float32),
        pltpu.VMEM((16,), jnp.float32),
        pltpu.SemaphoreType.DMA,
        pltpu.SemaphoreType.DMA,
    ],
)


def kernel(scores):
    out = _sc_kernel(scores)
    return out.reshape(_N, 16)[:, 0]


# speedup vs baseline: 1.1323x; 1.1323x over previous
import functools

import jax
import jax.numpy as jnp
from jax import lax
from jax.experimental import pallas as pl
from jax.experimental.pallas import tpu as pltpu
from jax.experimental.pallas import tpu_sc as plsc

_T = 1000
_N = 32
_NSTATE = 64
_NTRANS = 5
_C = _NSTATE * _NTRANS
_CH = 125
_NCHUNK = _T // _CH
_NORM_EVERY = 5
_LN2 = 0.6931471805599453


def _pgather(v, idxv):
    return v.at[idxv].get(mode="promise_in_bounds")


def _lane_bcast_max(v, iota):
    for sh in (8, 4, 2, 1):
        rot = (iota + sh) & 15
        v = jnp.maximum(v, _pgather(v, rot))
    return v


def _lane_bcast_sum(v, iota):
    for sh in (8, 4, 2, 1):
        rot = (iota + sh) & 15
        v = v + _pgather(v, rot)
    return v


def _sc_scan(scores_hbm, out_hbm, buf, out_v, sem0, sem1):
    wid = lax.axis_index("c") * 16 + lax.axis_index("s")
    iota = lax.iota(jnp.int32, 16)
    idxp = [4 * g + (iota >> 2) for g in range(4)]
    colv = [[80 * g + 5 * iota + j for j in range(_NTRANS)] for g in range(4)]

    sems = (sem0, sem1)

    def chunk_src(c):
        return scores_hbm.at[pl.ds(c * _CH, _CH), wid]

    def step(t, p, mb):
        tv = jnp.full((16,), t, jnp.int32)
        newp = []
        for g in range(4):
            acc = None
            for j in range(_NTRANS):
                m = plsc.load_gather(mb, [tv, colv[g][j]])
                pv = p[g] if j == 0 else _pgather(p[j - 1], idxp[g])
                term = jnp.exp(m) * pv
                acc = term if acc is None else acc + term
            newp.append(acc)
        return newp

    def renorm(p, cvec):
        m = jnp.maximum(jnp.maximum(p[0], p[1]), jnp.maximum(p[2], p[3]))
        m = _lane_bcast_max(m, iota)
        k = (plsc.bitcast(m, jnp.int32) >> 23) - 127
        scale = plsc.bitcast((127 - k) << 23, jnp.float32)
        return [pg * scale for pg in p], cvec + k

    def body(i, carry, mb):
        p = list(carry[:4])
        cvec = carry[4]
        for u in range(_NORM_EVERY):
            p = step(i * _NORM_EVERY + u, p, mb)
        p, cvec = renorm(p, cvec)
        return (*p, cvec)

    ones = jnp.ones((16,), jnp.float32)
    carry = (ones, ones, ones, ones, jnp.zeros((16,), jnp.int32))

    pltpu.async_copy(chunk_src(0), buf.at[0], sems[0])

    def chunk_pair(cc, carry):
        for par in range(2):
            c = 2 * cc + par
            cn = jnp.minimum(c + 1, _NCHUNK - 1)
            pltpu.async_copy(
                chunk_src(cn), buf.at[(par + 1) % 2], sems[(par + 1) % 2]
            )
            pltpu.make_async_copy(
                chunk_src(c), buf.at[par], sems[par]
            ).wait()
            carry = lax.fori_loop(
                0, _CH // _NORM_EVERY,
                functools.partial(body, mb=buf.at[par]), carry, unroll=False,
            )
        return carry

    carry = lax.fori_loop(0, _NCHUNK // 2, chunk_pair, carry, unroll=False)
    pltpu.make_async_copy(chunk_src(_NCHUNK - 1), buf.at[0], sems[0]).wait()

    p = carry[:4]
    cvec = carry[4]
    s = (p[0] + p[1]) + (p[2] + p[3])
    s = _lane_bcast_sum(s, iota)
    k2 = (plsc.bitcast(s, jnp.int32) >> 23) - 127
    mant = s * plsc.bitcast((127 - k2) << 23, jnp.float32)
    u = mant - 1.0
    z = u / (u + 2.0)
    z2 = z * z
    y = 2.0 * z * (1.0 + z2 * (1.0 / 3.0 + z2 * (0.2 + z2 * (1.0 / 7.0))))
    y = y + mant * jnp.exp(-y) - 1.0
    y = y + mant * jnp.exp(-y) - 1.0
    logz = (cvec + k2).astype(jnp.float32) * _LN2 + y
    out_v[...] = logz
    pltpu.sync_copy(out_v, out_hbm.at[pl.ds(wid * 16, 16)])


_sc_kernel = pl.kernel(
    _sc_scan,
    out_type=jax.ShapeDtypeStruct((_N * 16,), jnp.float32),
    mesh=plsc.VectorSubcoreMesh(core_axis_name="c", subcore_axis_name="s"),
    compiler_params=pltpu.CompilerParams(
        use_tc_tiling_on_sc=True, needs_layout_passes=False
    ),
    scratch_types=[
        pltpu.VMEM((2, _CH, _C), jnp.float32),
        pltpu.VMEM((16,), jnp.float32),
        pltpu.SemaphoreType.DMA,
        pltpu.SemaphoreType.DMA,
    ],
)


def kernel(scores):
    out = _sc_kernel(scores)
    return out.reshape(_N, 16)[:, 0]
